# Initial kernel scaffold; baseline (speedup 1.0000x reference)
import jax
import jax.numpy as jnp
from jax.experimental import pallas as pl


def kernel(z, t, conditioning, mask, params):
    raise NotImplementedError("write your pallas kernel here")


# speedup vs baseline: 34.0673x; 34.0673x over previous
import functools

import jax
import jax.numpy as jnp
from jax.experimental import pallas as pl
from jax.experimental.pallas import tpu as pltpu
from jax.experimental.pallas import tpu_sc as plsc

_N = 2048
_K = 20
_DH = 64
_DT = 32
_DC = _DT + 8
_GP = 128
_NB = 256


def _prep_body(t_ref, c_ref, w0, b0, w1, b1, w2, b2, embw, embb,
               cond_out, h0_out):
    t = t_ref[0, 0]
    half = _DT // 2
    lane = jax.lax.broadcasted_iota(jnp.int32, (1, _DT), 1).astype(jnp.float32)
    hidx = jnp.where(lane < half, lane, lane - half)
    freqs = jnp.exp(-jnp.log(10000.0) * hidx / (half - 1))
    args = t * freqs
    temb = jnp.where(lane < half, jnp.sin(args), jnp.cos(args))
    bsz = c_ref.shape[0]
    cin = jnp.concatenate(
        [jnp.broadcast_to(temb, (bsz, _DT)), c_ref[...]], axis=-1)
    f32 = jnp.float32
    h1 = jax.nn.silu(jnp.dot(cin, w0[...], preferred_element_type=f32) + b0[...])
    h2 = jax.nn.silu(jnp.dot(h1, w1[...], preferred_element_type=f32) + b1[...])
    cond_out[...] = jnp.dot(h2, w2[...], preferred_element_type=f32) + b2[...]
    h0_out[...] = embw[...] + embb[...]


def _prep(t, conditioning, params):
    bsz = conditioning.shape[0]
    full = lambda s: pl.BlockSpec(s, lambda: tuple(0 for _ in s))
    args = (t.reshape(1, 1), conditioning,
            params["cond_W"][0], params["cond_b"][0].reshape(1, -1),
            params["cond_W"][1], params["cond_b"][1].reshape(1, -1),
            params["cond_W"][2], params["cond_b"][2].reshape(1, -1),
            params["emb_W"], params["emb_b"].reshape(1, -1))
    return pl.pallas_call(
        _prep_body,
        out_shape=(jax.ShapeDtypeStruct((bsz, _DC), jnp.float32),
                   jax.ShapeDtypeStruct((1, _DH), jnp.float32)),
        in_specs=[full(a.shape) for a in args],
        out_specs=(full((bsz, _DC)), full((1, _DH))),
    )(*args)


def _knn_body(x_ref, xt_ref, idx_ref):
    xb = x_ref[0]
    n = xt_ref.shape[2]
    d2 = jnp.zeros((_NB, n), jnp.float32)
    for c in range(3):
        dc = xb[:, c:c + 1] - xt_ref[0, c:c + 1, :]
        d2 = d2 + dc * dc
    i = pl.program_id(1)
    rowg = jax.lax.broadcasted_iota(jnp.int32, (_NB, n), 0) + i * _NB
    colg = jax.lax.broadcasted_iota(jnp.int32, (_NB, n), 1)
    big = jnp.float32(1e10)
    d2 = jnp.where(rowg == colg, big, d2)
    for k in range(_K):
        mn = jnp.min(d2, axis=1, keepdims=True)
        am = jnp.min(jnp.where(d2 <= mn, colg, n), axis=1, keepdims=True)
        idx_ref[0, :, k:k + 1] = am
        d2 = jnp.where(colg == am, big, d2)


def _knn(z, zt):
    bsz, n, _ = z.shape
    return pl.pallas_call(
        _knn_body,
        grid=(bsz, n // _NB),
        in_specs=[
            pl.BlockSpec((1, _NB, 3), lambda b, i: (b, i, 0)),
            pl.BlockSpec((1, 3, n), lambda b, i: (b, 0, 0)),
        ],
        out_specs=pl.BlockSpec((1, _NB, _K), lambda b, i: (b, i, 0)),
        out_shape=jax.ShapeDtypeStruct((bsz, n, _K), jnp.int32),
    )(z, zt)


def _gather_rows(data, flat_idx):
    e = flat_idx.shape[1]
    w = 128
    mesh = plsc.VectorSubcoreMesh(core_axis_name="c", subcore_axis_name="s")

    @functools.partial(
        pl.kernel,
        out_type=jax.ShapeDtypeStruct((e, data.shape[1]), data.dtype),
        mesh=mesh)
    def run(x_hbm, i_hbm, o_hbm):
        def body(i_vmem, o_vmem):
            pltpu.sync_copy(x_hbm.at[i_vmem.at[0]], o_vmem)

        pltpu.emit_pipeline(
            body,
            grid=(e // w,),
            in_specs=[pl.BlockSpec((1, w), lambda i: (0, i))],
            out_specs=[pl.BlockSpec((w, data.shape[1]), lambda i: (i, 0))],
            core_axis_name=("c", "s"),
            dimension_semantics=(pltpu.PARALLEL,),
        )(i_hbm, o_hbm)

    return run(data, flat_idx)


def _layer_body(is_last, cat_ref, g_ref, cond_ref, z_ref,
                we0hi, we0hj, we0d2, we0c, be0, we1, be1,
                wx0, bx0, wx1r, bx1, wh0h, wh0m, bh0, wh1, bh1,
                out_ref):
    f32 = jnp.float32
    h = cat_ref[0, :, 0:_DH]
    x = cat_ref[0, :, _DH:_DH + 3]
    g = g_ref[0]
    hj = g[:, :, 0:_DH]
    xj = g[:, :, _DH:_DH + 3]
    diff = x[None] - xj
    d2 = jnp.sum(diff * diff, axis=-1, keepdims=True)

    pre_i = (jnp.dot(h, we0hi[...], preferred_element_type=f32)
             + jnp.dot(cond_ref[0], we0c[...], preferred_element_type=f32)
             + be0[...])
    e = _K * _NB
    pre = (jnp.dot(hj.reshape(e, _DH), we0hj[...],
                   preferred_element_type=f32).reshape(_K, _NB, _DH)
           + pre_i[None] + d2 * we0d2[...])
    m1 = jax.nn.silu(pre)
    m = jax.nn.silu(
        jnp.dot(m1.reshape(e, _DH), we1[...], preferred_element_type=f32)
        + be1[...])
    mx = jax.nn.silu(jnp.dot(m, wx0[...], preferred_element_type=f32)
                     + bx0[...])
    wv = (jnp.sum(mx * wx1r[...], axis=-1, keepdims=True)
          + bx1[0, 0])
    trans = diff * wv.reshape(_K, _NB, 1)
    aggx = jnp.sum(trans, axis=0) * jnp.float32(1.0 / _K)
    x_new = x + aggx
    if is_last:
        out_ref[0] = x_new - z_ref[0]
    else:
        aggm = jnp.sum(m.reshape(_K, _NB, _DH), axis=0)
        hin = jax.nn.silu(jnp.dot(h, wh0h[...], preferred_element_type=f32)
                          + jnp.dot(aggm, wh0m[...], preferred_element_type=f32)
                          + bh0[...])
        h_new = h + jnp.dot(hin, wh1[...], preferred_element_type=f32) + bh1[...]
        out_ref[0, :, 0:_DH] = h_new
        out_ref[0, :, _DH:_DH + 3] = x_new
        out_ref[0, :, _DH + 3:] = jnp.zeros((_NB, _GP - _DH - 3), f32)


def _layer(is_last, cat, g4, cond3, z, layer):
    bsz, n = cat.shape[0], cat.shape[1]
    we0 = layer["We"][0]
    weights = (we0[0:_DH], we0[_DH:2 * _DH], we0[2 * _DH:2 * _DH + 1],
               we0[2 * _DH + 1:], layer["be"][0].reshape(1, -1),
               layer["We"][1], layer["be"][1].reshape(1, -1),
               layer["Wx"][0], layer["bx"][0].reshape(1, -1),
               layer["Wx"][1].reshape(1, -1), layer["bx"][1].reshape(1, 1),
               layer["Wh"][0][0:_DH], layer["Wh"][0][_DH:],
               layer["bh"][0].reshape(1, -1),
               layer["Wh"][1], layer["bh"][1].reshape(1, -1))
    wspecs = [pl.BlockSpec(w.shape, lambda b, i: (0, 0)) for w in weights]
    in_specs = [
        pl.BlockSpec((1, _NB, _GP), lambda b, i: (b, i, 0)),
        pl.BlockSpec((1, _K, _NB, _GP), lambda b, i: (b, 0, i, 0)),
        pl.BlockSpec((1, 1, _DC), lambda b, i: (b, 0, 0)),
    ]
    args = [cat, g4, cond3]
    if is_last:
        in_specs.append(pl.BlockSpec((1, _NB, 3), lambda b, i: (b, i, 0)))
        args.append(z)
        out_spec = pl.BlockSpec((1, _NB, 3), lambda b, i: (b, i, 0))
        out_shape = jax.ShapeDtypeStruct((bsz, n, 3), jnp.float32)

        def body(*refs):
            return _layer_body(True, *refs)
    else:
        out_spec = pl.BlockSpec((1, _NB, _GP), lambda b, i: (b, i, 0))
        out_shape = jax.ShapeDtypeStruct((bsz, n, _GP), jnp.float32)

        def body(cat_ref, g_ref, cond_ref, *rest):
            return _layer_body(False, cat_ref, g_ref, cond_ref, None, *rest)
    return pl.pallas_call(
        body,
        grid=(bsz, n // _NB),
        in_specs=in_specs + wspecs,
        out_specs=out_spec,
        out_shape=out_shape,
    )(*args, *weights)


def kernel(z, t, conditioning, mask, params):
    bsz, n, _ = z.shape
    cond, h0 = _prep(t, conditioning, params)
    zt = jnp.transpose(z, (0, 2, 1))
    idx = _knn(z, zt)
    idxt = jnp.transpose(idx, (0, 2, 1))
    offs = (jnp.arange(bsz, dtype=jnp.int32) * n)[:, None, None]
    flat = (idxt + offs).reshape(1, bsz * _K * n)
    h0b = jnp.broadcast_to(h0[None], (bsz, n, _DH))
    cat = jnp.concatenate(
        [h0b, z, jnp.zeros((bsz, n, _GP - _DH - 3), jnp.float32)], axis=-1)
    cond3 = cond.reshape(bsz, 1, _DC)
    nl = len(params["layers"])
    for li, layer in enumerate(params["layers"]):
        g = _gather_rows(cat.reshape(bsz * n, _GP), flat)
        g4 = g.reshape(bsz, _K, n, _GP)
        if li < nl - 1:
            cat = _layer(False, cat, g4, cond3, None, layer)
        else:
            score = _layer(True, cat, g4, cond3, z, layer)
    return score


# speedup vs baseline: 35.0343x; 1.0284x over previous
import functools

import jax
import jax.numpy as jnp
from jax.experimental import pallas as pl
from jax.experimental.pallas import tpu as pltpu
from jax.experimental.pallas import tpu_sc as plsc

_N = 2048
_K = 20
_DH = 64
_DT = 32
_DC = _DT + 8
_GP = 128
_NB = 256


def _prep_body(t_ref, c_ref, w0, b0, w1, b1, w2, b2, embw, embb,
               cond_out, h0_out):
    t = t_ref[0, 0]
    half = _DT // 2
    lane = jax.lax.broadcasted_iota(jnp.int32, (1, _DT), 1).astype(jnp.float32)
    hidx = jnp.where(lane < half, lane, lane - half)
    freqs = jnp.exp(-jnp.log(10000.0) * hidx / (half - 1))
    args = t * freqs
    temb = jnp.where(lane < half, jnp.sin(args), jnp.cos(args))
    bsz = c_ref.shape[0]
    cin = jnp.concatenate(
        [jnp.broadcast_to(temb, (bsz, _DT)), c_ref[...]], axis=-1)
    f32 = jnp.float32
    h1 = jax.nn.silu(jnp.dot(cin, w0[...], preferred_element_type=f32) + b0[...])
    h2 = jax.nn.silu(jnp.dot(h1, w1[...], preferred_element_type=f32) + b1[...])
    cond_out[...] = jnp.dot(h2, w2[...], preferred_element_type=f32) + b2[...]
    h0_out[...] = embw[...] + embb[...]


def _prep(t, conditioning, params):
    bsz = conditioning.shape[0]
    full = lambda s: pl.BlockSpec(s, lambda: tuple(0 for _ in s))
    args = (t.reshape(1, 1), conditioning,
            params["cond_W"][0], params["cond_b"][0].reshape(1, -1),
            params["cond_W"][1], params["cond_b"][1].reshape(1, -1),
            params["cond_W"][2], params["cond_b"][2].reshape(1, -1),
            params["emb_W"], params["emb_b"].reshape(1, -1))
    return pl.pallas_call(
        _prep_body,
        out_shape=(jax.ShapeDtypeStruct((bsz, _DC), jnp.float32),
                   jax.ShapeDtypeStruct((1, _DH), jnp.float32)),
        in_specs=[full(a.shape) for a in args],
        out_specs=(full((bsz, _DC)), full((1, _DH))),
    )(*args)


def _knn_body(x_ref, xt_ref, idx_ref):
    xb = x_ref[0]
    n = xt_ref.shape[2]
    d2 = jnp.zeros((_NB, n), jnp.float32)
    for c in range(3):
        dc = xb[:, c:c + 1] - xt_ref[0, c:c + 1, :]
        d2 = d2 + dc * dc
    i = pl.program_id(1)
    rowg = jax.lax.broadcasted_iota(jnp.int32, (_NB, n), 0) + i * _NB
    colg = jax.lax.broadcasted_iota(jnp.int32, (_NB, n), 1)
    big = jnp.float32(1e10)
    d2 = jnp.where(rowg == colg, big, d2)
    for k in range(_K):
        mn = jnp.min(d2, axis=1, keepdims=True)
        am = jnp.min(jnp.where(d2 <= mn, colg, n), axis=1, keepdims=True)
        idx_ref[0, :, k:k + 1] = am
        d2 = jnp.where(colg == am, big, d2)


def _knn(z, zt):
    bsz, n, _ = z.shape
    return pl.pallas_call(
        _knn_body,
        grid=(bsz, n // _NB),
        in_specs=[
            pl.BlockSpec((1, _NB, 3), lambda b, i: (b, i, 0)),
            pl.BlockSpec((1, 3, n), lambda b, i: (b, 0, 0)),
        ],
        out_specs=pl.BlockSpec((1, _NB, _K), lambda b, i: (b, i, 0)),
        out_shape=jax.ShapeDtypeStruct((bsz, n, _K), jnp.int32),
    )(z, zt)


def _gather_rows(data, flat_idx):
    e = flat_idx.shape[1]
    w = 128
    mesh = plsc.VectorSubcoreMesh(core_axis_name="c", subcore_axis_name="s")

    @functools.partial(
        pl.kernel,
        out_type=jax.ShapeDtypeStruct((e, data.shape[1]), data.dtype),
        mesh=mesh)
    def run(x_hbm, i_hbm, o_hbm):
        def body(i_vmem, o_vmem):
            pltpu.sync_copy(x_hbm.at[i_vmem.at[0]], o_vmem)

        pltpu.emit_pipeline(
            body,
            grid=(e // w,),
            in_specs=[pl.BlockSpec((1, w), lambda i: (0, i))],
            out_specs=[pl.BlockSpec((w, data.shape[1]), lambda i: (i, 0))],
            core_axis_name=("c", "s"),
            dimension_semantics=(pltpu.PARALLEL,),
        )(i_hbm, o_hbm)

    return run(data, flat_idx)


def _layer_body(is_last, cat_ref, g_ref, cond_ref, z_ref,
                we0hi, we0hj, wd2, we0c, be0, we1, be1,
                wx0, bx0, wx1, bx1, wh0h, wh0m, bh0, wh1, bh1,
                out_ref):
    f32 = jnp.float32
    h = cat_ref[0, :, 0:_DH]
    x = cat_ref[0, :, _DH:_DH + 3]
    g = g_ref[0]
    hj = g[:, :, 0:_DH]
    xj = g[:, :, _DH:_DH + 3]
    diff = x[None] - xj
    sq = diff * diff
    e = _K * _NB

    pre_i = (jnp.dot(h, we0hi[...], preferred_element_type=f32)
             + jnp.dot(cond_ref[0], we0c[...], preferred_element_type=f32)
             + be0[...])
    pre = (jnp.dot(hj.reshape(e, _DH), we0hj[...],
                   preferred_element_type=f32)
           + jnp.dot(sq.reshape(e, 3), wd2[...],
                     preferred_element_type=f32)).reshape(_K, _NB, _DH)
    pre = pre + pre_i[None]
    m1 = jax.nn.silu(pre)
    m = jax.nn.silu(
        jnp.dot(m1.reshape(e, _DH), we1[...], preferred_element_type=f32)
        + be1[...])
    mx = jax.nn.silu(jnp.dot(m, wx0[...], preferred_element_type=f32)
                     + bx0[...])
    wv = (jnp.dot(mx, wx1[...], preferred_element_type=f32)
          + bx1[0, 0])
    trans = diff * wv.reshape(_K, _NB, 1)
    aggx = jnp.sum(trans, axis=0) * jnp.float32(1.0 / _K)
    x_new = x + aggx
    if is_last:
        out_ref[0] = x_new - z_ref[0]
    else:
        aggm = jnp.sum(m.reshape(_K, _NB, _DH), axis=0)
        hin = jax.nn.silu(jnp.dot(h, wh0h[...], preferred_element_type=f32)
                          + jnp.dot(aggm, wh0m[...], preferred_element_type=f32)
                          + bh0[...])
        h_new = h + jnp.dot(hin, wh1[...], preferred_element_type=f32) + bh1[...]
        out_ref[0, :, 0:_DH] = h_new
        out_ref[0, :, _DH:_DH + 3] = x_new
        out_ref[0, :, _DH + 3:] = jnp.zeros((_NB, _GP - _DH - 3), f32)


def _layer(is_last, cat, g4, cond3, z, layer):
    bsz, n = cat.shape[0], cat.shape[1]
    we0 = layer["We"][0]
    weights = (we0[0:_DH], we0[_DH:2 * _DH],
               jnp.broadcast_to(we0[2 * _DH:2 * _DH + 1], (3, _DH)),
               we0[2 * _DH + 1:], layer["be"][0].reshape(1, -1),
               layer["We"][1], layer["be"][1].reshape(1, -1),
               layer["Wx"][0], layer["bx"][0].reshape(1, -1),
               layer["Wx"][1], layer["bx"][1].reshape(1, 1),
               layer["Wh"][0][0:_DH], layer["Wh"][0][_DH:],
               layer["bh"][0].reshape(1, -1),
               layer["Wh"][1], layer["bh"][1].reshape(1, -1))
    wspecs = [pl.BlockSpec(w.shape, lambda b, i: (0, 0)) for w in weights]
    in_specs = [
        pl.BlockSpec((1, _NB, _GP), lambda b, i: (b, i, 0)),
        pl.BlockSpec((1, _K, _NB, _GP), lambda b, i: (b, 0, i, 0)),
        pl.BlockSpec((1, 1, _DC), lambda b, i: (b, 0, 0)),
    ]
    args = [cat, g4, cond3]
    if is_last:
        in_specs.append(pl.BlockSpec((1, _NB, 3), lambda b, i: (b, i, 0)))
        args.append(z)
        out_spec = pl.BlockSpec((1, _NB, 3), lambda b, i: (b, i, 0))
        out_shape = jax.ShapeDtypeStruct((bsz, n, 3), jnp.float32)

        def body(*refs):
            return _layer_body(True, *refs)
    else:
        out_spec = pl.BlockSpec((1, _NB, _GP), lambda b, i: (b, i, 0))
        out_shape = jax.ShapeDtypeStruct((bsz, n, _GP), jnp.float32)

        def body(cat_ref, g_ref, cond_ref, *rest):
            return _layer_body(False, cat_ref, g_ref, cond_ref, None, *rest)
    return pl.pallas_call(
        body,
        grid=(bsz, n // _NB),
        in_specs=in_specs + wspecs,
        out_specs=out_spec,
        out_shape=out_shape,
    )(*args, *weights)


def kernel(z, t, conditioning, mask, params):
    bsz, n, _ = z.shape
    cond, h0 = _prep(t, conditioning, params)
    zt = jnp.transpose(z, (0, 2, 1))
    idx = _knn(z, zt)
    idxt = jnp.transpose(idx, (0, 2, 1))
    offs = (jnp.arange(bsz, dtype=jnp.int32) * n)[:, None, None]
    flat = (idxt + offs).reshape(1, bsz * _K * n)
    h0b = jnp.broadcast_to(h0[None], (bsz, n, _DH))
    cat = jnp.concatenate(
        [h0b, z, jnp.zeros((bsz, n, _GP - _DH - 3), jnp.float32)], axis=-1)
    cond3 = cond.reshape(bsz, 1, _DC)
    nl = len(params["layers"])
    for li, layer in enumerate(params["layers"]):
        g = _gather_rows(cat.reshape(bsz * n, _GP), flat)
        g4 = g.reshape(bsz, _K, n, _GP)
        if li < nl - 1:
            cat = _layer(False, cat, g4, cond3, None, layer)
        else:
            score = _layer(True, cat, g4, cond3, z, layer)
    return score
